# baseline (speedup 1.0000x reference)
import functools

import jax
import jax.numpy as jnp
from jax import lax
from jax.experimental import pallas as pl
from jax.experimental.pallas import tpu as pltpu
from jax.experimental.pallas import tpu_sc as plsc

D = 64
NC, NS = 2, 16
NW = NC * NS
CHN = 20096
CH = 128


def _sc_mesh():
    return plsc.VectorSubcoreMesh(core_axis_name="c", subcore_axis_name="s",
                                  num_cores=NC, num_subcores=NS)


def _sc_gather_sweep(ut, mt, ut_tail, mt_tail, uid, mid, batch):
    n_users = ut.shape[1]
    n_movies = mt.shape[1]
    ut_tn = ut_tail.shape[1]
    mt_tn = mt_tail.shape[1]
    f32 = jnp.float32
    nvec = batch // 16

    @functools.partial(
        pl.kernel,
        out_type=(
            jax.ShapeDtypeStruct((D, batch), f32),
            jax.ShapeDtypeStruct((D, batch), f32),
        ),
        mesh=_sc_mesh(),
        compiler_params=pltpu.CompilerParams(needs_layout_passes=False),
        scratch_types=[
            pltpu.VMEM((batch,), jnp.int32),
            pltpu.VMEM((2, batch), f32),
            pltpu.VMEM((2, CHN), f32),
            pltpu.VMEM((2, CHN), f32),
            pltpu.VMEM((2, ut_tn), f32),
            pltpu.VMEM((2, mt_tn), f32),
            pltpu.SemaphoreType.DMA,
            pltpu.SemaphoreType.DMA,
        ],
    )
    def body(ut_hbm, mt_hbm, utt_hbm, mtt_hbm, uid_hbm, mid_hbm,
             uo_hbm, mo_hbm, ids_v, out_v, buf_a, buf_b, utail_v, mtail_v,
             sem_a, sem_b):
        wid = lax.axis_index("s") * NC + lax.axis_index("c")
        f0 = wid * 2

        def scan_chunk(ref, cbase, csize):
            usize = jnp.uint32(csize)

            @plsc.parallel_loop(0, nvec, unroll=12)
            def _(i):
                idv = ids_v[pl.ds(i * 16, 16)]
                local = idv - cbase
                m = plsc.bitcast(local, jnp.uint32) < usize
                pos = lax.iota(jnp.int32, 16) + i * 16
                zero = jnp.zeros((16,), jnp.int32)
                one = jnp.ones((16,), jnp.int32)
                g0 = plsc.load_gather(ref, [zero, local], mask=m)
                g1 = plsc.load_gather(ref, [one, local], mask=m)
                plsc.store_scatter(out_v, [zero, pos], g0, mask=m)
                plsc.store_scatter(out_v, [one, pos], g1, mask=m)

        def sweep_table(tab_hbm, tail_hbm, tail_v, tail_n,
                        ids_hbm, o_hbm, n_rows):
            pltpu.sync_copy(ids_hbm, ids_v)
            aligned = n_rows - tail_n
            n_full = aligned // CHN
            rem = aligned - n_full * CHN
            rows = pl.ds(f0, 2)

            def start(cidx, buf, sem):
                pltpu.async_copy(
                    tab_hbm.at[rows, pl.ds(cidx * CHN, CHN)], buf, sem)

            def wait(buf, sem):
                pltpu.make_async_copy(
                    tab_hbm.at[rows, pl.ds(0, CHN)], buf, sem).wait()

            npairs = (n_full - 1) // 2
            leftover = n_full - 2 * npairs
            start(0, buf_a, sem_a)

            def pair_it(k, carry):
                c0 = 2 * k
                start(c0 + 1, buf_b, sem_b)
                wait(buf_a, sem_a)
                scan_chunk(buf_a, c0 * CHN, CHN)
                start(c0 + 2, buf_a, sem_a)
                wait(buf_b, sem_b)
                scan_chunk(buf_b, (c0 + 1) * CHN, CHN)
                return carry

            lax.fori_loop(0, npairs, pair_it, 0)
            c0 = 2 * npairs
            rem_started = False
            if leftover == 2:
                start(c0 + 1, buf_b, sem_b)
                wait(buf_a, sem_a)
                scan_chunk(buf_a, c0 * CHN, CHN)
                if rem:
                    pltpu.async_copy(
                        tab_hbm.at[rows, pl.ds(n_full * CHN, rem)],
                        buf_a.at[:, pl.ds(0, rem)], sem_a)
                    rem_started = True
                wait(buf_b, sem_b)
                scan_chunk(buf_b, (c0 + 1) * CHN, CHN)
            else:
                if rem:
                    pltpu.async_copy(
                        tab_hbm.at[rows, pl.ds(n_full * CHN, rem)],
                        buf_b.at[:, pl.ds(0, rem)], sem_b)
                wait(buf_a, sem_a)
                scan_chunk(buf_a, c0 * CHN, CHN)
            if rem:
                rbuf, rsem = (buf_a, sem_a) if rem_started else (buf_b, sem_b)
                pltpu.make_async_copy(
                    tab_hbm.at[rows, pl.ds(0, rem)],
                    rbuf.at[:, pl.ds(0, rem)], rsem).wait()
                scan_chunk(rbuf, n_full * CHN, rem)
            if tail_n:
                pltpu.sync_copy(tail_hbm.at[pl.ds(f0, 2)], tail_v)
                scan_chunk(tail_v, aligned, tail_n)
            pltpu.sync_copy(out_v, o_hbm.at[pl.ds(f0, 2)])

        sweep_table(ut_hbm, utt_hbm, utail_v, ut_tn, uid_hbm, uo_hbm, n_users)
        sweep_table(mt_hbm, mtt_hbm, mtail_v, mt_tn, mid_hbm, mo_hbm, n_movies)

    return body(ut, mt, ut_tail, mt_tail, uid, mid)


def _sc_gather_bias(ub1d, mb1d, uid, mid, batch):
    bpw = batch // NW
    nch = bpw // CH
    f32 = jnp.float32

    @functools.partial(
        pl.kernel,
        out_type=(
            jax.ShapeDtypeStruct((batch,), f32),
            jax.ShapeDtypeStruct((batch,), f32),
        ),
        mesh=_sc_mesh(),
        compiler_params=pltpu.CompilerParams(use_tc_tiling_on_sc=False),
        scratch_types=[
            pltpu.VMEM((nch, CH), jnp.int32),
            pltpu.VMEM((nch, CH), jnp.int32),
            pltpu.VMEM((bpw,), f32),
            pltpu.VMEM((bpw,), f32),
            pltpu.SemaphoreType.DMA,
        ],
    )
    def body(ub_hbm, mb_hbm, uid_hbm, mid_hbm, ubo_hbm, mbo_hbm,
             uidx, midx, ubv, mbv, sem):
        wid = lax.axis_index("s") * NC + lax.axis_index("c")
        base = wid * bpw
        for j in range(nch):
            pltpu.sync_copy(uid_hbm.at[pl.ds(base + j * CH, CH)], uidx.at[j])
            pltpu.sync_copy(mid_hbm.at[pl.ds(base + j * CH, CH)], midx.at[j])
        copies = []
        for j in range(nch):
            sl = pl.ds(j * CH, CH)
            copies.append(pltpu.async_copy(ub_hbm.at[uidx.at[j]], ubv.at[sl], sem))
            copies.append(pltpu.async_copy(mb_hbm.at[midx.at[j]], mbv.at[sl], sem))
        for c in copies:
            c.wait()
        out_sl = pl.ds(base, bpw)
        pltpu.sync_copy(ubv, ubo_hbm.at[out_sl])
        pltpu.sync_copy(mbv, mbo_hbm.at[out_sl])

    return body(ub1d, mb1d, uid, mid)


def _ln_t(x, w, b, eps=1e-5):
    mean = jnp.mean(x, axis=0, keepdims=True)
    xc = x - mean
    var = jnp.mean(xc * xc, axis=0, keepdims=True)
    return xc / jnp.sqrt(var + eps) * w + b


def _tc_body(ut_ref, mt_ref, ub_ref, mb_ref, aid_ref,
             af_ref, unw, unb, mnw, mnb, anw, anb, gb_ref, out_ref):
    blk = ut_ref.shape[1]
    u = _ln_t(jnp.maximum(ut_ref[...], 0.0), unw[...], unb[...])
    m = _ln_t(jnp.maximum(mt_ref[...], 0.0), mnw[...], mnb[...])
    af_t = jnp.transpose(af_ref[...])
    a_tab_t = _ln_t(jnp.maximum(af_t, 0.0), anw[...], anb[...])
    n_age = af_ref.shape[0]
    aid = aid_ref[...].reshape(1, blk)
    onehot = (aid == lax.broadcasted_iota(jnp.int32, (n_age, blk), 0)
              ).astype(jnp.float32)
    ages = jnp.dot(a_tab_t, onehot, preferred_element_type=jnp.float32)
    dot = jnp.sum(u * m * ages, axis=0, keepdims=True)
    preds = dot * 0.125 + ub_ref[...].reshape(1, blk) \
        + mb_ref[...].reshape(1, blk) + gb_ref[...]
    out_ref[...] = jnp.clip(preds, -0.1, 1.1).reshape(1, 1, blk)


def _tc_compute(u_t, m_t, ub3, mb3, aid3, age_factors,
                unw, unb, mnw, mnb, anw, anb, gb, batch, grid):
    blk = batch // grid
    n_age = age_factors.shape[0]
    row_spec = pl.BlockSpec((D, blk), lambda i: (0, i))
    vec_spec = pl.BlockSpec((1, 1, blk), lambda i: (i, 0, 0))
    par_spec = pl.BlockSpec((D, 1), lambda i: (0, 0))
    return pl.pallas_call(
        _tc_body,
        grid=(grid,),
        in_specs=[
            row_spec, row_spec, vec_spec, vec_spec, vec_spec,
            pl.BlockSpec((n_age, D), lambda i: (0, 0)),
            par_spec, par_spec, par_spec, par_spec, par_spec, par_spec,
            pl.BlockSpec((1, 1), lambda i: (0, 0)),
        ],
        out_specs=vec_spec,
        out_shape=jax.ShapeDtypeStruct((grid, 1, blk), jnp.float32),
    )(u_t, m_t, ub3, mb3, aid3, age_factors,
      unw, unb, mnw, mnb, anw, anb, gb)


def kernel(user_ids, movie_ids, age_bucket_ids,
           user_factors, movie_factors, age_factors,
           user_norm_w, user_norm_b, movie_norm_w, movie_norm_b,
           age_norm_w, age_norm_b, user_bias, movie_bias, global_bias):
    batch = user_ids.shape[0]
    grid = 8
    blk = batch // grid
    uid = user_ids.astype(jnp.int32)
    mid = movie_ids.astype(jnp.int32)
    nu = user_factors.shape[0]
    nm = movie_factors.shape[0]
    ua = (nu // 128) * 128
    ma = (nm // 128) * 128
    u_t, m_t = _sc_gather_sweep(
        user_factors.T, movie_factors.T,
        user_factors[ua:, :].T, movie_factors[ma:, :].T,
        uid, mid, batch)
    ub, mb = _sc_gather_bias(user_bias.reshape(-1), movie_bias.reshape(-1),
                             uid, mid, batch)
    preds = _tc_compute(
        u_t, m_t, ub.reshape(grid, 1, blk), mb.reshape(grid, 1, blk),
        age_bucket_ids.astype(jnp.int32).reshape(grid, 1, blk), age_factors,
        user_norm_w.reshape(D, 1), user_norm_b.reshape(D, 1),
        movie_norm_w.reshape(D, 1), movie_norm_b.reshape(D, 1),
        age_norm_w.reshape(D, 1), age_norm_b.reshape(D, 1),
        global_bias.reshape(1, 1), batch, grid)
    return preds.reshape(batch)


# speedup vs baseline: 1.1929x; 1.1929x over previous
import functools

import jax
import jax.numpy as jnp
from jax import lax
from jax.experimental import pallas as pl
from jax.experimental.pallas import tpu as pltpu
from jax.experimental.pallas import tpu_sc as plsc

D = 64
NC, NS = 2, 16
NW = NC * NS
CHN = 20096
CH = 128


def _sc_mesh():
    return plsc.VectorSubcoreMesh(core_axis_name="c", subcore_axis_name="s",
                                  num_cores=NC, num_subcores=NS)


def _sc_gather_sweep(ut, mt, ut_tail, mt_tail, uid, mid, batch):
    n_users = ut.shape[1]
    n_movies = mt.shape[1]
    ut_tn = ut_tail.shape[1]
    mt_tn = mt_tail.shape[1]
    f32 = jnp.float32
    nvec = batch // 16

    @functools.partial(
        pl.kernel,
        out_type=(
            jax.ShapeDtypeStruct((D, batch), f32),
            jax.ShapeDtypeStruct((D, batch), f32),
        ),
        mesh=_sc_mesh(),
        compiler_params=pltpu.CompilerParams(needs_layout_passes=False),
        scratch_types=[
            pltpu.VMEM((batch,), jnp.int32),
            pltpu.VMEM((2, batch), f32),
            pltpu.VMEM((2, CHN), f32),
            pltpu.VMEM((2, CHN), f32),
            pltpu.VMEM((2, ut_tn), f32),
            pltpu.VMEM((2, mt_tn), f32),
            pltpu.SemaphoreType.DMA,
            pltpu.SemaphoreType.DMA,
        ],
    )
    def body(ut_hbm, mt_hbm, utt_hbm, mtt_hbm, uid_hbm, mid_hbm,
             uo_hbm, mo_hbm, ids_v, out_v, buf_a, buf_b, utail_v, mtail_v,
             sem_a, sem_b):
        wid = lax.axis_index("s") * NC + lax.axis_index("c")
        f0 = wid * 2

        def scan_chunk(ref, cbase, csize):
            usize = jnp.uint32(csize)

            @plsc.parallel_loop(0, nvec, unroll=8)
            def _(i):
                idv = ids_v[pl.ds(i * 16, 16)]
                local = idv - cbase
                m = plsc.bitcast(local, jnp.uint32) < usize
                pos = lax.iota(jnp.int32, 16) + i * 16
                zero = jnp.zeros((16,), jnp.int32)
                one = jnp.ones((16,), jnp.int32)
                g0 = plsc.load_gather(ref, [zero, local], mask=m)
                g1 = plsc.load_gather(ref, [one, local], mask=m)
                plsc.store_scatter(out_v, [zero, pos], g0, mask=m)
                plsc.store_scatter(out_v, [one, pos], g1, mask=m)

        def sweep_table(tab_hbm, tail_hbm, tail_v, tail_n,
                        ids_hbm, o_hbm, n_rows):
            pltpu.sync_copy(ids_hbm, ids_v)
            aligned = n_rows - tail_n
            n_full = aligned // CHN
            rem = aligned - n_full * CHN
            rows = pl.ds(f0, 2)

            def start(cidx, buf, sem):
                pltpu.async_copy(
                    tab_hbm.at[rows, pl.ds(cidx * CHN, CHN)], buf, sem)

            def wait(buf, sem):
                pltpu.make_async_copy(
                    tab_hbm.at[rows, pl.ds(0, CHN)], buf, sem).wait()

            npairs = (n_full - 1) // 2
            leftover = n_full - 2 * npairs
            start(0, buf_a, sem_a)

            def pair_it(k, carry):
                c0 = 2 * k
                start(c0 + 1, buf_b, sem_b)
                wait(buf_a, sem_a)
                scan_chunk(buf_a, c0 * CHN, CHN)
                start(c0 + 2, buf_a, sem_a)
                wait(buf_b, sem_b)
                scan_chunk(buf_b, (c0 + 1) * CHN, CHN)
                return carry

            lax.fori_loop(0, npairs, pair_it, 0)
            c0 = 2 * npairs
            rem_started = False
            if leftover == 2:
                start(c0 + 1, buf_b, sem_b)
                wait(buf_a, sem_a)
                scan_chunk(buf_a, c0 * CHN, CHN)
                if rem:
                    pltpu.async_copy(
                        tab_hbm.at[rows, pl.ds(n_full * CHN, rem)],
                        buf_a.at[:, pl.ds(0, rem)], sem_a)
                    rem_started = True
                wait(buf_b, sem_b)
                scan_chunk(buf_b, (c0 + 1) * CHN, CHN)
            else:
                if rem:
                    pltpu.async_copy(
                        tab_hbm.at[rows, pl.ds(n_full * CHN, rem)],
                        buf_b.at[:, pl.ds(0, rem)], sem_b)
                wait(buf_a, sem_a)
                scan_chunk(buf_a, c0 * CHN, CHN)
            if rem:
                rbuf, rsem = (buf_a, sem_a) if rem_started else (buf_b, sem_b)
                pltpu.make_async_copy(
                    tab_hbm.at[rows, pl.ds(0, rem)],
                    rbuf.at[:, pl.ds(0, rem)], rsem).wait()
                scan_chunk(rbuf, n_full * CHN, rem)
            if tail_n:
                pltpu.sync_copy(tail_hbm.at[pl.ds(f0, 2)], tail_v)
                scan_chunk(tail_v, aligned, tail_n)
            pltpu.sync_copy(out_v, o_hbm.at[pl.ds(f0, 2)])

        sweep_table(ut_hbm, utt_hbm, utail_v, ut_tn, uid_hbm, uo_hbm, n_users)
        sweep_table(mt_hbm, mtt_hbm, mtail_v, mt_tn, mid_hbm, mo_hbm, n_movies)

    return body(ut, mt, ut_tail, mt_tail, uid, mid)


def _sc_gather_bias(ub1d, mb1d, uid, mid, batch):
    bpw = batch // NW
    nch = bpw // CH
    f32 = jnp.float32

    @functools.partial(
        pl.kernel,
        out_type=(
            jax.ShapeDtypeStruct((batch,), f32),
            jax.ShapeDtypeStruct((batch,), f32),
        ),
        mesh=_sc_mesh(),
        compiler_params=pltpu.CompilerParams(use_tc_tiling_on_sc=False),
        scratch_types=[
            pltpu.VMEM((nch, CH), jnp.int32),
            pltpu.VMEM((nch, CH), jnp.int32),
            pltpu.VMEM((bpw,), f32),
            pltpu.VMEM((bpw,), f32),
            pltpu.SemaphoreType.DMA,
        ],
    )
    def body(ub_hbm, mb_hbm, uid_hbm, mid_hbm, ubo_hbm, mbo_hbm,
             uidx, midx, ubv, mbv, sem):
        wid = lax.axis_index("s") * NC + lax.axis_index("c")
        base = wid * bpw
        for j in range(nch):
            pltpu.sync_copy(uid_hbm.at[pl.ds(base + j * CH, CH)], uidx.at[j])
            pltpu.sync_copy(mid_hbm.at[pl.ds(base + j * CH, CH)], midx.at[j])
        copies = []
        for j in range(nch):
            sl = pl.ds(j * CH, CH)
            copies.append(pltpu.async_copy(ub_hbm.at[uidx.at[j]], ubv.at[sl], sem))
            copies.append(pltpu.async_copy(mb_hbm.at[midx.at[j]], mbv.at[sl], sem))
        for c in copies:
            c.wait()
        out_sl = pl.ds(base, bpw)
        pltpu.sync_copy(ubv, ubo_hbm.at[out_sl])
        pltpu.sync_copy(mbv, mbo_hbm.at[out_sl])

    return body(ub1d, mb1d, uid, mid)


def _ln_t(x, w, b, eps=1e-5):
    mean = jnp.mean(x, axis=0, keepdims=True)
    xc = x - mean
    var = jnp.mean(xc * xc, axis=0, keepdims=True)
    return xc / jnp.sqrt(var + eps) * w + b


def _tc_body(ut_ref, mt_ref, ub_ref, mb_ref, aid_ref,
             af_ref, unw, unb, mnw, mnb, anw, anb, gb_ref, out_ref):
    blk = ut_ref.shape[1]
    u = _ln_t(jnp.maximum(ut_ref[...], 0.0), unw[...], unb[...])
    m = _ln_t(jnp.maximum(mt_ref[...], 0.0), mnw[...], mnb[...])
    af_t = jnp.transpose(af_ref[...])
    a_tab_t = _ln_t(jnp.maximum(af_t, 0.0), anw[...], anb[...])
    n_age = af_ref.shape[0]
    aid = aid_ref[...].reshape(1, blk)
    onehot = (aid == lax.broadcasted_iota(jnp.int32, (n_age, blk), 0)
              ).astype(jnp.float32)
    ages = jnp.dot(a_tab_t, onehot, preferred_element_type=jnp.float32)
    dot = jnp.sum(u * m * ages, axis=0, keepdims=True)
    preds = dot * 0.125 + ub_ref[...].reshape(1, blk) \
        + mb_ref[...].reshape(1, blk) + gb_ref[...]
    out_ref[...] = jnp.clip(preds, -0.1, 1.1).reshape(1, 1, blk)


def _tc_compute(u_t, m_t, ub3, mb3, aid3, age_factors,
                unw, unb, mnw, mnb, anw, anb, gb, batch, grid):
    blk = batch // grid
    n_age = age_factors.shape[0]
    row_spec = pl.BlockSpec((D, blk), lambda i: (0, i))
    vec_spec = pl.BlockSpec((1, 1, blk), lambda i: (i, 0, 0))
    par_spec = pl.BlockSpec((D, 1), lambda i: (0, 0))
    return pl.pallas_call(
        _tc_body,
        grid=(grid,),
        in_specs=[
            row_spec, row_spec, vec_spec, vec_spec, vec_spec,
            pl.BlockSpec((n_age, D), lambda i: (0, 0)),
            par_spec, par_spec, par_spec, par_spec, par_spec, par_spec,
            pl.BlockSpec((1, 1), lambda i: (0, 0)),
        ],
        out_specs=vec_spec,
        out_shape=jax.ShapeDtypeStruct((grid, 1, blk), jnp.float32),
    )(u_t, m_t, ub3, mb3, aid3, age_factors,
      unw, unb, mnw, mnb, anw, anb, gb)


def kernel(user_ids, movie_ids, age_bucket_ids,
           user_factors, movie_factors, age_factors,
           user_norm_w, user_norm_b, movie_norm_w, movie_norm_b,
           age_norm_w, age_norm_b, user_bias, movie_bias, global_bias):
    batch = user_ids.shape[0]
    grid = 8
    blk = batch // grid
    uid = user_ids.astype(jnp.int32)
    mid = movie_ids.astype(jnp.int32)
    nu = user_factors.shape[0]
    nm = movie_factors.shape[0]
    ua = (nu // 128) * 128
    ma = (nm // 128) * 128
    u_t, m_t = _sc_gather_sweep(
        user_factors.T, movie_factors.T,
        user_factors[ua:, :].T, movie_factors[ma:, :].T,
        uid, mid, batch)
    ub, mb = _sc_gather_bias(user_bias.reshape(-1), movie_bias.reshape(-1),
                             uid, mid, batch)
    preds = _tc_compute(
        u_t, m_t, ub.reshape(grid, 1, blk), mb.reshape(grid, 1, blk),
        age_bucket_ids.astype(jnp.int32).reshape(grid, 1, blk), age_factors,
        user_norm_w.reshape(D, 1), user_norm_b.reshape(D, 1),
        movie_norm_w.reshape(D, 1), movie_norm_b.reshape(D, 1),
        age_norm_w.reshape(D, 1), age_norm_b.reshape(D, 1),
        global_bias.reshape(1, 1), batch, grid)
    return preds.reshape(batch)
